# baseline (speedup 1.0000x reference)
import functools

import numpy as np
import jax
import jax.numpy as jnp
from jax import lax
from jax.experimental import pallas as pl
from jax.experimental.pallas import tpu as pltpu
from jax.experimental.pallas import tpu_sc as plsc

_C, _H, _W, _B, _DIM = 3, 64, 64, 32, 64
_SEQ = _C * _H * _W


def _source_perm() -> np.ndarray:
    diagonals = [[] for _ in range(_H + _W - 1)]
    for i in range(_H):
        for j in range(_W):
            s = i + j
            if s % 2 == 0:
                diagonals[s].insert(0, (i, j))
            else:
                diagonals[s].append((i, j))
    triples = []
    for diag in diagonals:
        for ij in diag:
            for c in range(_C):
                triples.append((c,) + ij)
    a = np.array(triples, dtype=np.int64)
    d_of_k = (a[:, 0] * _H + a[:, 1]) * _W + a[:, 2]
    src = np.empty(_SEQ, dtype=np.int32)
    src[d_of_k] = np.arange(_SEQ, dtype=np.int32)
    return src


_SRC = _source_perm()

_NW = 32
_D_PER_W = _SEQ // _NW
_DBLK = 16
_NTILE = _D_PER_W // _DBLK
_NBUF = 3
_AHEAD = 2


@functools.partial(
    pl.kernel,
    out_type=jax.ShapeDtypeStruct((_B, _SEQ, _DIM), jnp.float32),
    mesh=plsc.VectorSubcoreMesh(core_axis_name="c", subcore_axis_name="s"),
    scratch_types=[
        pltpu.VMEM((_D_PER_W,), jnp.int32),
        pltpu.VMEM((_NBUF, _DBLK, _B, _DIM), jnp.float32),
        pltpu.SemaphoreType.DMA((_NBUF,)),
        pltpu.SemaphoreType.DMA((_NBUF,)),
    ],
    compiler_params=pltpu.CompilerParams(use_tc_tiling_on_sc=False),
)
def _zigzag_gather(x_hbm, idx_hbm, out_hbm, idx_v, bufs_v, gsem, ssem):
    w = lax.axis_index("s") * 2 + lax.axis_index("c")
    d0 = w * _D_PER_W

    pltpu.sync_copy(idx_hbm.at[pl.ds(d0, _D_PER_W)], idx_v)

    def gather(t, p):
        pltpu.async_copy(
            x_hbm.at[idx_v.at[pl.ds(t * _DBLK, _DBLK)]],
            bufs_v.at[p],
            gsem.at[p],
        )

    def wait_gather(p):
        pltpu.make_async_copy(
            x_hbm.at[pl.ds(0, _DBLK)], bufs_v.at[p], gsem.at[p]
        ).wait()

    def stores(t, p):
        base = d0 + t * _DBLK
        for b in range(_B):
            pltpu.async_copy(
                bufs_v.at[p, :, b, :],
                out_hbm.at[b, pl.ds(base, _DBLK), :],
                ssem.at[p],
            )

    def drain_stores(p):
        pltpu.make_async_copy(
            x_hbm.at[pl.ds(0, _DBLK)], bufs_v.at[p], ssem.at[p]
        ).wait()

    for t in range(_AHEAD):
        gather(t, t % _NBUF)

    ngroup = _NTILE // _NBUF

    def body(g, carry):
        for b in range(_NBUF):
            pn = (b + _AHEAD) % _NBUF
            if b < _NBUF - _AHEAD:
                @pl.when(g > 0)
                def _():
                    drain_stores(pn)

                gather(g * _NBUF + b + _AHEAD, pn)
            else:
                @pl.when(g < ngroup - 1)
                def _():
                    drain_stores(pn)
                    gather(g * _NBUF + b + _AHEAD, pn)

            wait_gather(b)
            stores(g * _NBUF + b, b)
        return carry

    lax.fori_loop(0, ngroup, body, 0)

    for p in range(_NBUF):
        drain_stores(p)


def kernel(x):
    idx = jnp.asarray(_SRC)
    out = _zigzag_gather(x, idx)
    return out.reshape(_B, _C, _H, _W, _DIM)


# speedup vs baseline: 1.4293x; 1.4293x over previous
import functools

import numpy as np
import jax
import jax.numpy as jnp
from jax import lax
from jax.experimental import pallas as pl
from jax.experimental.pallas import tpu as pltpu
from jax.experimental.pallas import tpu_sc as plsc

_C, _H, _W, _B, _DIM = 3, 64, 64, 32, 64
_SEQ = _C * _H * _W


def _source_perm() -> np.ndarray:
    diagonals = [[] for _ in range(_H + _W - 1)]
    for i in range(_H):
        for j in range(_W):
            s = i + j
            if s % 2 == 0:
                diagonals[s].insert(0, (i, j))
            else:
                diagonals[s].append((i, j))
    triples = []
    for diag in diagonals:
        for ij in diag:
            for c in range(_C):
                triples.append((c,) + ij)
    a = np.array(triples, dtype=np.int64)
    d_of_k = (a[:, 0] * _H + a[:, 1]) * _W + a[:, 2]
    src = np.empty(_SEQ, dtype=np.int32)
    src[d_of_k] = np.arange(_SEQ, dtype=np.int32)
    return src


_SRC = _source_perm()

_NW = 32
_D_PER_W = _SEQ // _NW
_DBLK = 8
_NTILE = _D_PER_W // _DBLK
_NBUF = 3
_AHEAD = 2


@functools.partial(
    pl.kernel,
    out_type=jax.ShapeDtypeStruct((_B, _SEQ, _DIM), jnp.float32),
    mesh=plsc.VectorSubcoreMesh(core_axis_name="c", subcore_axis_name="s"),
    scratch_types=[
        pltpu.VMEM((_D_PER_W + 8,), jnp.int32),
        pltpu.VMEM((_NBUF, _DBLK, _B, _DIM), jnp.float32),
        pltpu.SemaphoreType.DMA((_NBUF,)),
        pltpu.SemaphoreType.DMA((_NBUF,)),
    ],
    compiler_params=pltpu.CompilerParams(use_tc_tiling_on_sc=True),
)
def _zigzag_gather(x_hbm, idx_hbm, out_hbm, idx_v, bufs_v, gsem, ssem):
    w = lax.axis_index("s") * 2 + lax.axis_index("c")
    d0 = w * _D_PER_W

    pltpu.sync_copy(idx_hbm.at[pl.ds(d0, _D_PER_W)], idx_v.at[pl.ds(0, _D_PER_W)])

    def gather(t, p):
        sds = idx_v[pl.ds(t * _DBLK, 16)]
        for i in range(_DBLK):
            pltpu.async_copy(
                x_hbm.at[pl.ds(sds[i], 1), :, :],
                bufs_v.at[p, pl.ds(i, 1), :, :],
                gsem.at[p],
            )

    def wait_gather(p):
        pltpu.make_async_copy(
            x_hbm.at[pl.ds(0, _DBLK)], bufs_v.at[p], gsem.at[p]
        ).wait()

    def stores(t, p):
        base = d0 + t * _DBLK
        for b in range(_B):
            pltpu.async_copy(
                bufs_v.at[p, :, b, :],
                out_hbm.at[b, pl.ds(base, _DBLK), :],
                ssem.at[p],
            )

    def drain_stores(p):
        pltpu.make_async_copy(
            x_hbm.at[pl.ds(0, _DBLK)], bufs_v.at[p], ssem.at[p]
        ).wait()

    for t in range(_AHEAD):
        gather(t, t % _NBUF)

    ngroup = _NTILE // _NBUF

    def body(g, carry):
        for b in range(_NBUF):
            pn = (b + _AHEAD) % _NBUF
            if b < _NBUF - _AHEAD:
                @pl.when(g > 0)
                def _():
                    drain_stores(pn)

                gather(g * _NBUF + b + _AHEAD, pn)
            else:
                @pl.when(g < ngroup - 1)
                def _():
                    drain_stores(pn)
                    gather(g * _NBUF + b + _AHEAD, pn)

            wait_gather(b)
            stores(g * _NBUF + b, b)
        return carry

    lax.fori_loop(0, ngroup, body, 0)

    for p in range(_NBUF):
        drain_stores(p)


def kernel(x):
    idx = jnp.asarray(_SRC)
    out = _zigzag_gather(x, idx)
    return out.reshape(_B, _C, _H, _W, _DIM)
